# baseline (speedup 1.0000x reference)
import jax
import jax.numpy as jnp
from jax import lax
from jax.experimental import pallas as pl
from jax.experimental.pallas import tpu as pltpu

_NUM_ANCHORS = 3
_NUM_CH = 85
_GH = 76
_GW = 76
_S = _GH * _GW
_STRIDE = 8.0
_ANCHOR_W = (10.0, 16.0, 33.0)
_ANCHOR_H = (13.0, 30.0, 23.0)


def _decode_half(v, a, h):
    gh2 = _GH // 2
    sub = v[:, h * gh2 : (h + 1) * gh2, :]

    gx = lax.broadcasted_iota(jnp.int32, (1, gh2, _GW), 2).astype(jnp.float32)
    gy = (
        lax.broadcasted_iota(jnp.int32, (1, gh2, _GW), 1).astype(jnp.float32)
        + h * gh2
    )

    aw = jnp.where(a == 0, _ANCHOR_W[0], jnp.where(a == 1, _ANCHOR_W[1], _ANCHOR_W[2]))
    ah = jnp.where(a == 0, _ANCHOR_H[0], jnp.where(a == 1, _ANCHOR_H[1], _ANCHOR_H[2]))

    r0 = (jax.nn.sigmoid(sub[0:1]) + gx) * _STRIDE
    r1 = (jax.nn.sigmoid(sub[1:2]) + gy) * _STRIDE
    r2 = jnp.exp(sub[2:3]) * aw
    r3 = jnp.exp(sub[3:4]) * ah
    rest = jax.nn.sigmoid(sub[4:])
    res = jnp.concatenate([r0, r1, r2, r3, rest], axis=0)
    return res.reshape(_NUM_CH, _S // 2).T


def _decode_body(x_ref, o_ref):
    a = pl.program_id(1)
    h = pl.program_id(2)
    v = x_ref[0]

    @pl.when(h == 0)
    def _():
        o_ref[0] = _decode_half(v, a, 0)

    @pl.when(h == 1)
    def _():
        o_ref[0] = _decode_half(v, a, 1)


def kernel(x, target):
    del target
    B = x.shape[0]

    output = pl.pallas_call(
        _decode_body,
        grid=(B, _NUM_ANCHORS, 2),
        in_specs=[
            pl.BlockSpec((1, _NUM_CH, _GH, _GW), lambda b, a, h: (b, a, 0, 0))
        ],
        out_specs=pl.BlockSpec(
            (1, _S // 2, _NUM_CH), lambda b, a, h: (b, 2 * a + h, 0)
        ),
        out_shape=jax.ShapeDtypeStruct((B, _NUM_ANCHORS * _S, _NUM_CH), jnp.float32),
        compiler_params=pltpu.CompilerParams(
            dimension_semantics=("parallel", "arbitrary", "arbitrary")
        ),
    )(x)

    zero = jnp.zeros((1,), dtype=jnp.float32)
    return (output, zero, zero, zero, zero, zero)


# speedup vs baseline: 1.2251x; 1.2251x over previous
import jax
import jax.numpy as jnp
from jax import lax
from jax.experimental import pallas as pl
from jax.experimental.pallas import tpu as pltpu

_NUM_ANCHORS = 3
_NUM_CH = 85
_GH = 76
_GW = 76
_S = _GH * _GW
_STRIDE = 8.0
_ANCHOR_W = (10.0, 16.0, 33.0)
_ANCHOR_H = (13.0, 30.0, 23.0)


def _decode_body(x_ref, o_ref):
    a = pl.program_id(1)
    v = x_ref[0]

    gx = lax.broadcasted_iota(jnp.int32, (1, _GH, _GW), 2).astype(jnp.float32)
    gy = lax.broadcasted_iota(jnp.int32, (1, _GH, _GW), 1).astype(jnp.float32)

    aw = jnp.where(a == 0, _ANCHOR_W[0], jnp.where(a == 1, _ANCHOR_W[1], _ANCHOR_W[2]))
    ah = jnp.where(a == 0, _ANCHOR_H[0], jnp.where(a == 1, _ANCHOR_H[1], _ANCHOR_H[2]))

    r0 = (jax.nn.sigmoid(v[0:1]) + gx) * _STRIDE
    r1 = (jax.nn.sigmoid(v[1:2]) + gy) * _STRIDE
    r2 = jnp.exp(v[2:3]) * aw
    r3 = jnp.exp(v[3:4]) * ah
    rest = jax.nn.sigmoid(v[4:])
    res = jnp.concatenate([r0, r1, r2, r3, rest], axis=0)

    o_ref[0] = res.reshape(_NUM_CH, _S).T


def kernel(x, target):
    del target
    B = x.shape[0]

    output = pl.pallas_call(
        _decode_body,
        grid=(B, _NUM_ANCHORS),
        in_specs=[pl.BlockSpec((1, _NUM_CH, _GH, _GW), lambda b, a: (b, a, 0, 0))],
        out_specs=pl.BlockSpec((1, _S, _NUM_CH), lambda b, a: (b, a, 0)),
        out_shape=jax.ShapeDtypeStruct((B, _NUM_ANCHORS * _S, _NUM_CH), jnp.float32),
        compiler_params=pltpu.CompilerParams(
            dimension_semantics=("parallel", "arbitrary")
        ),
    )(x)

    zero = jnp.zeros((1,), dtype=jnp.float32)
    return (output, zero, zero, zero, zero, zero)
